# baseline (speedup 1.0000x reference)
import functools

import jax
import jax.numpy as jnp
import numpy as np
from jax import lax
from jax.experimental import pallas as pl
from jax.experimental.pallas import tpu as pltpu
from jax.experimental.pallas import tpu_sc as plsc

NUM_EMB = 8192
DIM = 256
B_TOTAL = 32 * 1024
M_TILE = 512
N_STEPS = B_TOTAL // M_TILE
COMMITMENT_COST = 0.25

N_CHUNK = 8
CN = NUM_EMB // N_CHUNK


def _lane_weights():
    j = np.arange(NUM_EMB)
    w2 = np.zeros((NUM_EMB, NUM_EMB // 128), np.float32)
    w2[j, j // 128] = np.exp2(64.0 - (j % 128))
    return jnp.asarray(w2, dtype=jnp.bfloat16)


def _argmin_body(x_ref, e_ref, idx_ref, mm_ref, c_ref, mmax_ref):
    i = pl.program_id(0)
    cur = lax.rem(i, 2)
    prv = lax.rem(i + 1, 2)

    x = x_ref[...]
    c_ref[cur] = jnp.sum(x * x, axis=1, keepdims=True)
    mmax = jnp.full((M_TILE, 1), -jnp.inf, jnp.float32)
    for k in range(N_CHUNK):
        e = e_ref[pl.ds(k * CN, CN), :]
        mm_k = lax.dot_general(x, e, (((1,), (1,)), ((), ())),
                               preferred_element_type=jnp.float32)
        mm_ref[cur, :, pl.ds(k * CN, CN)] = mm_k
        mmax = jnp.maximum(mmax, jnp.max(mm_k, axis=1, keepdims=True))
    mmax_ref[cur] = mmax

    c = c_ref[prv]
    dmin = c - 2.0 * mmax_ref[prv]
    bits = lax.bitcast_convert_type(dmin, jnp.int32)
    nxt = lax.bitcast_convert_type(bits + 1, jnp.float32)
    thr = ((c - dmin) - 0.5 * (nxt - dmin)) * 0.5
    mm = mm_ref[prv]
    cols = lax.broadcasted_iota(
        jnp.int32, (M_TILE, NUM_EMB), 1).astype(jnp.float32)
    fidx = jnp.min(jnp.where(mm >= thr, cols, float(NUM_EMB)), axis=1,
                   keepdims=True)
    idx_ref[...] = fidx.astype(jnp.int32)


def _st_body(x_ref, q_ref, st_ref, ssum_ref):
    i = pl.program_id(0)
    x = x_ref[...]
    d = q_ref[...] - x
    st_ref[...] = x + d

    @pl.when(i == 0)
    def _():
        ssum_ref[0, 0] = 0.0

    ssum_ref[0, 0] += jnp.sum(d * d)


@functools.lru_cache(maxsize=None)
def _make_gather_st(n_rows=B_TOTAL, row_base=0):
    info = plsc.get_sparse_core_info()
    nc, ns = info.num_cores, info.num_subcores
    nw = nc * ns
    b_per_w = n_rows // nw
    rpc = 64
    chunks = b_per_w // rpc
    groups = DIM // 16
    mesh = plsc.VectorSubcoreMesh(core_axis_name="c", subcore_axis_name="s")

    @functools.partial(
        pl.kernel, mesh=mesh,
        out_type=(
            jax.ShapeDtypeStruct((n_rows, DIM), jnp.float32),
            jax.ShapeDtypeStruct((nw, 16), jnp.float32),
        ),
        scratch_types=[
            pltpu.VMEM((chunks, rpc), jnp.int32),
            pltpu.VMEM((2, rpc, DIM), jnp.float32),
            pltpu.VMEM((2, rpc, DIM), jnp.float32),
            pltpu.VMEM((16,), jnp.float32),
            pltpu.SemaphoreType.DMA,
            pltpu.SemaphoreType.DMA,
            pltpu.SemaphoreType.DMA,
            pltpu.SemaphoreType.DMA,
        ],
    )
    def gather_st_k(table_hbm, idx_hbm, x_hbm, st_hbm, part_hbm,
                    idx_v, q_v, x_v, acc_v, sq0, sq1, sx0, sx1):
        wid = lax.axis_index("s") * nc + lax.axis_index("c")
        pltpu.sync_copy(idx_hbm.at[pl.ds(wid * chunks, chunks), :], idx_v)
        sq = (sq0, sq1)
        sx = (sx0, sx1)

        def issue(j):
            b = j % 2
            base = wid * b_per_w + j * rpc
            hq = pltpu.async_copy(table_hbm.at[idx_v.at[j]], q_v.at[b],
                                  sq[b])
            hx = pltpu.async_copy(x_hbm.at[pl.ds(row_base + base, rpc), :],
                                  x_v.at[b], sx[b])
            return hq, hx

        acc = jnp.zeros((16,), jnp.float32)
        pend = issue(0)
        for j in range(chunks):
            b = j % 2
            hq, hx = pend
            if j + 1 < chunks:
                pend = issue(j + 1)
            hq.wait()
            hx.wait()

            def row_body(r, a, _b=b):
                for g in range(groups):
                    sl = pl.ds(g * 16, 16)
                    q = q_v[_b, r, sl]
                    xx = x_v[_b, r, sl]
                    d = q - xx
                    q_v[_b, r, sl] = xx + d
                    a = a + d * d
                return a

            acc = lax.fori_loop(0, rpc, row_body, acc)
            base = wid * b_per_w + j * rpc
            pltpu.sync_copy(q_v.at[b], st_hbm.at[pl.ds(base, rpc), :])
        acc_v[...] = acc
        pltpu.sync_copy(acc_v, part_hbm.at[wid])

    return gather_st_k


N_SLICES = 2
S_ROWS = B_TOTAL // N_SLICES
S_STEPS = S_ROWS // M_TILE


def _argmin_slice(flat, embedding_weight, s):
    base_blk = s * S_STEPS
    return pl.pallas_call(
        _argmin_body,
        grid=(S_STEPS + 1,),
        in_specs=[
            pl.BlockSpec(
                (M_TILE, DIM),
                lambda i: (base_blk + jnp.minimum(i, S_STEPS - 1), 0)),
            pl.BlockSpec((NUM_EMB, DIM), lambda i: (0, 0)),
        ],
        out_specs=pl.BlockSpec((M_TILE, 1),
                               lambda i: (jnp.maximum(i - 1, 0), 0)),
        out_shape=jax.ShapeDtypeStruct((S_ROWS, 1), jnp.int32),
        scratch_shapes=[
            pltpu.VMEM((2, M_TILE, NUM_EMB), jnp.float32),
            pltpu.VMEM((2, M_TILE, 1), jnp.float32),
            pltpu.VMEM((2, M_TILE, 1), jnp.float32),
        ],
    )(flat, embedding_weight)


def kernel(inputs, embedding_weight):
    input_shape = inputs.shape
    flat = inputs.reshape(-1, DIM)

    sts, parts, idxs = [], [], []
    for s in range(N_SLICES):
        idx2d = _argmin_slice(flat, embedding_weight, s)
        idx_rows = idx2d.reshape(-1, 64)
        st_s, part_s = _make_gather_st(S_ROWS, s * S_ROWS)(
            embedding_weight, idx_rows, flat)
        idxs.append(idx2d)
        sts.append(st_s)
        parts.append(part_s)

    loss = COMMITMENT_COST * (
        sum(jnp.sum(p) for p in parts) / float(B_TOTAL * DIM))
    st = jnp.concatenate(sts, axis=0)
    idx_all = jnp.concatenate(idxs, axis=0)
    return (loss, st.reshape(input_shape), idx_all)


# speedup vs baseline: 1.0379x; 1.0379x over previous
import functools

import jax
import jax.numpy as jnp
from jax import lax
from jax.experimental import pallas as pl
from jax.experimental.pallas import tpu as pltpu
from jax.experimental.pallas import tpu_sc as plsc

NUM_EMB = 8192
DIM = 256
B_TOTAL = 32 * 1024
M_TILE = 512
N_STEPS = B_TOTAL // M_TILE
COMMITMENT_COST = 0.25

N_CHUNK = 8
CN = NUM_EMB // N_CHUNK


def _argmin_body(x_ref, e_ref, idx_ref, mm_ref, c_ref, mmax_ref):
    i = pl.program_id(0)
    cur = lax.rem(i, 2)
    prv = lax.rem(i + 1, 2)

    x = x_ref[...]
    c_ref[cur] = jnp.sum(x * x, axis=1, keepdims=True)
    mmax = jnp.full((M_TILE, 1), -jnp.inf, jnp.float32)
    for k in range(N_CHUNK):
        e = e_ref[pl.ds(k * CN, CN), :]
        mm_k = lax.dot_general(x, e, (((1,), (1,)), ((), ())),
                               preferred_element_type=jnp.float32)
        mm_ref[cur, :, pl.ds(k * CN, CN)] = mm_k
        mmax = jnp.maximum(mmax, jnp.max(mm_k, axis=1, keepdims=True))
    mmax_ref[cur] = mmax

    c = c_ref[prv]
    dmin = c - 2.0 * mmax_ref[prv]
    bits = lax.bitcast_convert_type(dmin, jnp.int32)
    nxt = lax.bitcast_convert_type(bits + 1, jnp.float32)
    thr = ((c - dmin) - 0.5 * (nxt - dmin)) * 0.5
    mm = mm_ref[prv]
    cols = lax.broadcasted_iota(
        jnp.int32, (M_TILE, NUM_EMB), 1).astype(jnp.float32)
    fidx = jnp.min(jnp.where(mm >= thr, cols, float(NUM_EMB)), axis=1,
                   keepdims=True)
    idx_ref[...] = fidx.astype(jnp.int32)


@functools.lru_cache(maxsize=None)
def _make_gather_st():
    info = plsc.get_sparse_core_info()
    nc, ns = info.num_cores, info.num_subcores
    nw = nc * ns
    b_per_w = B_TOTAL // nw
    chunks = 16
    rpc = b_per_w // chunks
    groups = DIM // 16
    mesh = plsc.VectorSubcoreMesh(core_axis_name="c", subcore_axis_name="s")

    @functools.partial(
        pl.kernel, mesh=mesh,
        out_type=(
            jax.ShapeDtypeStruct((B_TOTAL, DIM), jnp.float32),
            jax.ShapeDtypeStruct((nw, 16), jnp.float32),
        ),
        scratch_types=[
            pltpu.VMEM((chunks, rpc), jnp.int32),
            pltpu.VMEM((2, rpc, DIM), jnp.float32),
            pltpu.VMEM((2, rpc, DIM), jnp.float32),
            pltpu.VMEM((16,), jnp.float32),
            pltpu.SemaphoreType.DMA,
            pltpu.SemaphoreType.DMA,
            pltpu.SemaphoreType.DMA,
            pltpu.SemaphoreType.DMA,
        ],
    )
    def gather_st_k(table_hbm, idx_hbm, x_hbm, st_hbm, part_hbm,
                    idx_v, q_v, x_v, acc_v, sq0, sq1, sx0, sx1):
        wid = lax.axis_index("s") * nc + lax.axis_index("c")
        pltpu.sync_copy(idx_hbm.at[pl.ds(wid * chunks, chunks), :], idx_v)
        sq = (sq0, sq1)
        sx = (sx0, sx1)

        def issue(j):
            b = j % 2
            base = wid * b_per_w + j * rpc
            hq = pltpu.async_copy(table_hbm.at[idx_v.at[j]], q_v.at[b],
                                  sq[b])
            hx = pltpu.async_copy(x_hbm.at[pl.ds(base, rpc), :], x_v.at[b],
                                  sx[b])
            return hq, hx

        acc = jnp.zeros((16,), jnp.float32)
        pend = issue(0)
        for j in range(chunks):
            b = j % 2
            hq, hx = pend
            if j + 1 < chunks:
                pend = issue(j + 1)
            hq.wait()
            hx.wait()

            def row_body(r, a, _b=b):
                for g in range(groups):
                    sl = pl.ds(g * 16, 16)
                    q = q_v[_b, r, sl]
                    xx = x_v[_b, r, sl]
                    d = q - xx
                    q_v[_b, r, sl] = xx + d
                    a = a + d * d
                return a

            acc = lax.fori_loop(0, rpc, row_body, acc)
            base = wid * b_per_w + j * rpc
            pltpu.sync_copy(q_v.at[b], st_hbm.at[pl.ds(base, rpc), :])
        acc_v[...] = acc
        pltpu.sync_copy(acc_v, part_hbm.at[wid])

    return gather_st_k


def kernel(inputs, embedding_weight):
    input_shape = inputs.shape
    flat = inputs.reshape(-1, DIM)

    grid = (N_STEPS + 1,)
    idx2d = pl.pallas_call(
        _argmin_body,
        grid=grid,
        in_specs=[
            pl.BlockSpec((M_TILE, DIM),
                         lambda i: (jnp.minimum(i, N_STEPS - 1), 0)),
            pl.BlockSpec((NUM_EMB, DIM), lambda i: (0, 0)),
        ],
        out_specs=pl.BlockSpec((M_TILE, 1),
                               lambda i: (jnp.maximum(i - 1, 0), 0)),
        out_shape=jax.ShapeDtypeStruct((B_TOTAL, 1), jnp.int32),
        scratch_shapes=[
            pltpu.VMEM((2, M_TILE, NUM_EMB), jnp.float32),
            pltpu.VMEM((2, M_TILE, 1), jnp.float32),
            pltpu.VMEM((2, M_TILE, 1), jnp.float32),
        ],
    )(flat, embedding_weight)

    idx_rows = idx2d.reshape(-1, 64)
    st, partials = _make_gather_st()(embedding_weight, idx_rows, flat)

    loss = COMMITMENT_COST * (jnp.sum(partials) / float(B_TOTAL * DIM))
    return (loss, st.reshape(input_shape), idx2d)
